# initial kernel scaffold (unmeasured)
import jax
import jax.numpy as jnp
from jax import lax
from jax.experimental import pallas as pl
from jax.experimental.pallas import tpu as pltpu

N_DEV = 4


def kernel(x, w_mat, scale_x, scale_w):
    m_per, k = x.shape
    _, n_per = w_mat.shape

    x8 = x.astype(jnp.float8_e4m3fn)
    w8 = w_mat.astype(jnp.float8_e4m3fn)

    def body(x_ref, w_ref, sx_ref, sw_ref, out_ref, comm_ref, send_sems, recv_sems):
        my_pos = lax.axis_index("i")
        left = (my_pos - 1) % N_DEV
        right = (my_pos + 1) % N_DEV

        barrier_sem = pltpu.get_barrier_semaphore()
        for nbr in (left, right):
            pl.semaphore_signal(
                barrier_sem, inc=1,
                device_id=(nbr,), device_id_type=pl.DeviceIdType.MESH,
            )
        pl.semaphore_wait(barrier_sem, 2)

        scale = sx_ref[0] * sw_ref[0]

        comm_ref[0] = x_ref[...]
        out_ref[pl.ds(my_pos * m_per, m_per), :] = (
            jnp.dot(x_ref[...], w_ref[...], preferred_element_type=jnp.float32)
            * scale
        )

        for h in range(N_DEV - 1):
            rdma = pltpu.make_async_remote_copy(
                src_ref=comm_ref.at[h],
                dst_ref=comm_ref.at[h + 1],
                send_sem=send_sems.at[h],
                recv_sem=recv_sems.at[h],
                device_id=(right,),
                device_id_type=pl.DeviceIdType.MESH,
            )
            rdma.start()
            rdma.wait()
            origin = (my_pos - h - 1) % N_DEV
            out_ref[pl.ds(origin * m_per, m_per), :] = (
                jnp.dot(
                    comm_ref[h + 1], w_ref[...],
                    preferred_element_type=jnp.float32,
                )
                * scale
            )

    return pl.pallas_call(
        body,
        out_shape=jax.ShapeDtypeStruct((N_DEV * m_per, n_per), jnp.float32),
        in_specs=[
            pl.BlockSpec(memory_space=pltpu.VMEM),
            pl.BlockSpec(memory_space=pltpu.VMEM),
            pl.BlockSpec(memory_space=pltpu.SMEM),
            pl.BlockSpec(memory_space=pltpu.SMEM),
        ],
        out_specs=pl.BlockSpec(memory_space=pltpu.VMEM),
        scratch_shapes=[
            pltpu.VMEM((N_DEV, m_per, k), jnp.float8_e4m3fn),
            pltpu.SemaphoreType.DMA((N_DEV - 1,)),
            pltpu.SemaphoreType.DMA((N_DEV - 1,)),
        ],
        compiler_params=pltpu.CompilerParams(collective_id=0),
    )(x8, w8, scale_x, scale_w)


# baseline (device time: 242204 ns/iter reference)
import jax
import jax.numpy as jnp
from jax import lax
from jax.experimental import pallas as pl
from jax.experimental.pallas import tpu as pltpu

N_DEV = 4


def kernel(x, w_mat, scale_x, scale_w):
    m_per, k = x.shape
    _, n_per = w_mat.shape

    x8 = x.astype(jnp.float8_e4m3fn)
    w8 = w_mat.astype(jnp.float8_e4m3fn)

    def body(x_ref, w_ref, sx_ref, sw_ref, out_ref, comm_ref, send_sems, recv_sems):
        my_pos = lax.axis_index("i")
        left = (my_pos - 1) % N_DEV
        right = (my_pos + 1) % N_DEV

        barrier_sem = pltpu.get_barrier_semaphore()
        for nbr in (left, right):
            pl.semaphore_signal(
                barrier_sem, inc=1,
                device_id=(nbr,), device_id_type=pl.DeviceIdType.MESH,
            )
        pl.semaphore_wait(barrier_sem, 2)

        scale = sx_ref[0] * sw_ref[0]

        comm_ref[0] = x_ref[...]
        out_ref[pl.ds(my_pos * m_per, m_per), :] = (
            jnp.dot(x_ref[...], w_ref[...], preferred_element_type=jnp.float32)
            * scale
        )

        for h in range(N_DEV - 1):
            rdma = pltpu.make_async_remote_copy(
                src_ref=comm_ref.at[h],
                dst_ref=comm_ref.at[h + 1],
                send_sem=send_sems.at[h],
                recv_sem=recv_sems.at[h],
                device_id=(right,),
                device_id_type=pl.DeviceIdType.MESH,
            )
            rdma.start()
            rdma.wait()
            origin = (my_pos - h - 1) % N_DEV
            out_ref[pl.ds(origin * m_per, m_per), :] = (
                jnp.dot(
                    comm_ref[h + 1], w_ref[...],
                    preferred_element_type=jnp.float32,
                )
                * scale
            )

    return pl.pallas_call(
        body,
        out_shape=jax.ShapeDtypeStruct((N_DEV * m_per, n_per), jnp.float32),
        in_specs=[
            pl.BlockSpec(memory_space=pltpu.VMEM),
            pl.BlockSpec(memory_space=pltpu.VMEM),
            pl.BlockSpec(memory_space=pltpu.SMEM),
            pl.BlockSpec(memory_space=pltpu.SMEM),
        ],
        out_specs=pl.BlockSpec(memory_space=pltpu.VMEM),
        scratch_shapes=[
            pltpu.VMEM((N_DEV, m_per, k), jnp.float8_e4m3fn),
            pltpu.SemaphoreType.DMA((N_DEV - 1,)),
            pltpu.SemaphoreType.DMA((N_DEV - 1,)),
        ],
        compiler_params=pltpu.CompilerParams(
            collective_id=0,
            vmem_limit_bytes=100 * 1024 * 1024,
        ),
    )(x8, w8, scale_x, scale_w)


# device time: 147164 ns/iter; 1.6458x vs baseline; 1.6458x over previous
import jax
import jax.numpy as jnp
from jax import lax
from jax.experimental import pallas as pl
from jax.experimental.pallas import tpu as pltpu

N_DEV = 4


def kernel(x, w_mat, scale_x, scale_w):
    m_per, k = x.shape
    _, n_per = w_mat.shape
    h_per = m_per // 2

    x8 = x.astype(jnp.float8_e4m3fn)
    w8 = w_mat.astype(jnp.float8_e4m3fn)

    def body(x_ref, w_ref, sx_ref, sw_ref, out_ref,
             comm_r, comm_l, send_r, recv_r, send_l, recv_l):
        my_pos = lax.axis_index("i")
        left = (my_pos - 1) % N_DEV
        right = (my_pos + 1) % N_DEV

        barrier_sem = pltpu.get_barrier_semaphore()
        for nbr in (left, right):
            pl.semaphore_signal(
                barrier_sem, inc=1,
                device_id=(nbr,), device_id_type=pl.DeviceIdType.MESH,
            )
        pl.semaphore_wait(barrier_sem, 2)

        scale = sx_ref[0] * sw_ref[0]

        comm_r[0] = x_ref[:h_per]
        comm_l[0] = x_ref[h_per:]

        def gemm(rows_ref, row_start, nrows):
            out_ref[pl.ds(row_start, nrows), :] = (
                jnp.dot(rows_ref[...], w_ref[...],
                        preferred_element_type=jnp.float32)
                * scale
            )

        for h in range(N_DEV - 1):
            rdma_r = pltpu.make_async_remote_copy(
                src_ref=comm_r.at[h], dst_ref=comm_r.at[h + 1],
                send_sem=send_r.at[h], recv_sem=recv_r.at[h],
                device_id=(right,), device_id_type=pl.DeviceIdType.MESH,
            )
            rdma_l = pltpu.make_async_remote_copy(
                src_ref=comm_l.at[h], dst_ref=comm_l.at[h + 1],
                send_sem=send_l.at[h], recv_sem=recv_l.at[h],
                device_id=(left,), device_id_type=pl.DeviceIdType.MESH,
            )
            rdma_r.start()
            rdma_l.start()
            if h == 0:
                gemm(x_ref, my_pos * m_per, m_per)
            else:
                gemm(comm_r.at[h], ((my_pos - h) % N_DEV) * m_per, h_per)
                gemm(comm_l.at[h], ((my_pos + h) % N_DEV) * m_per + h_per, h_per)
            rdma_r.wait()
            rdma_l.wait()

        last = N_DEV - 1
        gemm(comm_r.at[last], ((my_pos - last) % N_DEV) * m_per, h_per)
        gemm(comm_l.at[last], ((my_pos + last) % N_DEV) * m_per + h_per, h_per)

    return pl.pallas_call(
        body,
        out_shape=jax.ShapeDtypeStruct((N_DEV * m_per, n_per), jnp.float32),
        in_specs=[
            pl.BlockSpec(memory_space=pltpu.VMEM),
            pl.BlockSpec(memory_space=pltpu.VMEM),
            pl.BlockSpec(memory_space=pltpu.SMEM),
            pl.BlockSpec(memory_space=pltpu.SMEM),
        ],
        out_specs=pl.BlockSpec(memory_space=pltpu.VMEM),
        scratch_shapes=[
            pltpu.VMEM((N_DEV, h_per, k), jnp.float8_e4m3fn),
            pltpu.VMEM((N_DEV, h_per, k), jnp.float8_e4m3fn),
            pltpu.SemaphoreType.DMA((N_DEV - 1,)),
            pltpu.SemaphoreType.DMA((N_DEV - 1,)),
            pltpu.SemaphoreType.DMA((N_DEV - 1,)),
            pltpu.SemaphoreType.DMA((N_DEV - 1,)),
        ],
        compiler_params=pltpu.CompilerParams(
            collective_id=0,
            vmem_limit_bytes=100 * 1024 * 1024,
        ),
    )(x8, w8, scale_x, scale_w)


# device time: 138278 ns/iter; 1.7516x vs baseline; 1.0643x over previous
import jax
import jax.numpy as jnp
from jax import lax
from jax.experimental import pallas as pl
from jax.experimental.pallas import tpu as pltpu

N_DEV = 4


def kernel(x, w_mat, scale_x, scale_w):
    m_per, k = x.shape
    _, n_per = w_mat.shape
    h_per = m_per // 2

    x8 = x.astype(jnp.float8_e4m3fn)
    w8 = w_mat.astype(jnp.float8_e4m3fn)

    def body(x_ref, w_ref, sx_ref, sw_ref, out_ref,
             comm_r, comm_l, stage, copy_sems,
             send_r, recv_r, send_l, recv_l):
        my_pos = lax.axis_index("i")
        left = (my_pos - 1) % N_DEV
        right = (my_pos + 1) % N_DEV

        comm_r[0] = x_ref[:h_per]
        comm_l[0] = x_ref[h_per:]

        barrier_sem = pltpu.get_barrier_semaphore()
        for nbr in (left, right):
            pl.semaphore_signal(
                barrier_sem, inc=1,
                device_id=(nbr,), device_id_type=pl.DeviceIdType.MESH,
            )
        pl.semaphore_wait(barrier_sem, 2)

        scale = sx_ref[0] * sw_ref[0]

        pending = [None, None]
        task_idx = [0]

        def task(src_ref, row_start):
            s = task_idx[0] % 2
            task_idx[0] += 1
            if pending[s] is not None:
                pending[s].wait()
            stage[s] = (
                jnp.dot(src_ref[...], w_ref[...],
                        preferred_element_type=jnp.float32)
                * scale
            )
            cp = pltpu.make_async_copy(
                stage.at[s],
                out_ref.at[pl.ds(row_start, h_per)],
                copy_sems.at[s],
            )
            cp.start()
            pending[s] = cp

        for h in range(N_DEV - 1):
            rdma_r = pltpu.make_async_remote_copy(
                src_ref=comm_r.at[h], dst_ref=comm_r.at[h + 1],
                send_sem=send_r.at[h], recv_sem=recv_r.at[h],
                device_id=(right,), device_id_type=pl.DeviceIdType.MESH,
            )
            rdma_l = pltpu.make_async_remote_copy(
                src_ref=comm_l.at[h], dst_ref=comm_l.at[h + 1],
                send_sem=send_l.at[h], recv_sem=recv_l.at[h],
                device_id=(left,), device_id_type=pl.DeviceIdType.MESH,
            )
            rdma_r.start()
            rdma_l.start()
            if h == 0:
                task(comm_r.at[0], my_pos * m_per)
                task(comm_l.at[0], my_pos * m_per + h_per)
            else:
                task(comm_r.at[h], ((my_pos - h) % N_DEV) * m_per)
                task(comm_l.at[h], ((my_pos + h) % N_DEV) * m_per + h_per)
            rdma_r.wait()
            rdma_l.wait()

        last = N_DEV - 1
        task(comm_r.at[last], ((my_pos - last) % N_DEV) * m_per)
        task(comm_l.at[last], ((my_pos + last) % N_DEV) * m_per + h_per)
        for cp in pending:
            cp.wait()

    return pl.pallas_call(
        body,
        out_shape=jax.ShapeDtypeStruct((N_DEV * m_per, n_per), jnp.float32),
        in_specs=[
            pl.BlockSpec(memory_space=pltpu.VMEM),
            pl.BlockSpec(memory_space=pltpu.VMEM),
            pl.BlockSpec(memory_space=pltpu.SMEM),
            pl.BlockSpec(memory_space=pltpu.SMEM),
        ],
        out_specs=pl.BlockSpec(memory_space=pl.ANY),
        scratch_shapes=[
            pltpu.VMEM((N_DEV, h_per, k), jnp.float8_e4m3fn),
            pltpu.VMEM((N_DEV, h_per, k), jnp.float8_e4m3fn),
            pltpu.VMEM((2, h_per, n_per), jnp.float32),
            pltpu.SemaphoreType.DMA((2,)),
            pltpu.SemaphoreType.DMA((N_DEV - 1,)),
            pltpu.SemaphoreType.DMA((N_DEV - 1,)),
            pltpu.SemaphoreType.DMA((N_DEV - 1,)),
            pltpu.SemaphoreType.DMA((N_DEV - 1,)),
        ],
        compiler_params=pltpu.CompilerParams(
            collective_id=0,
            vmem_limit_bytes=60 * 1024 * 1024,
        ),
    )(x8, w8, scale_x, scale_w)


# device time: 107672 ns/iter; 2.2495x vs baseline; 1.2843x over previous
import jax
import jax.numpy as jnp
from jax import lax
from jax.experimental import pallas as pl
from jax.experimental.pallas import tpu as pltpu

N_DEV = 4
W_PIECE = 256
SUBS = 4


def kernel(x, w_mat, scale_x, scale_w):
    m_per, k = x.shape
    _, n_per = w_mat.shape
    h_per = m_per // 2
    n_pieces = k // W_PIECE

    def body(x_ref, w_ref, sx_ref, sw_ref, out_ref,
             w8_ref, wbuf, xbuf, comm_r, comm_l, stage,
             wsems, xsems, copy_sems, send_r, recv_r, send_l, recv_l):
        my_pos = lax.axis_index("i")
        left = (my_pos - 1) % N_DEV
        right = (my_pos + 1) % N_DEV
        q_per = m_per // 4

        xcps = []
        for p in range(4):
            cp = pltpu.make_async_copy(
                x_ref.at[pl.ds(p * q_per, q_per)], xbuf.at[p], xsems.at[p],
            )
            cp.start()
            xcps.append(cp)

        barrier_sem = pltpu.get_barrier_semaphore()
        for nbr in (left, right):
            pl.semaphore_signal(
                barrier_sem, inc=1,
                device_id=(nbr,), device_id_type=pl.DeviceIdType.MESH,
            )
        pl.semaphore_wait(barrier_sem, 2)

        scale = sx_ref[0] * sw_ref[0]

        def w_piece_copy(p):
            return pltpu.make_async_copy(
                w_ref.at[pl.ds(p * W_PIECE, W_PIECE)],
                wbuf.at[p % 2],
                wsems.at[p % 2],
            )

        def stream_w():
            for p in range(n_pieces):
                if p + 1 < n_pieces:
                    w_piece_copy(p + 1).start()
                w_piece_copy(p).wait()
                w8_ref[pl.ds(p * W_PIECE, W_PIECE)] = (
                    wbuf[p % 2].astype(jnp.float8_e4m3fn)
                )

        pending = [None, None]
        task_idx = [0]

        def task(src_ref, row_start, nrows):
            s = task_idx[0] % 2
            task_idx[0] += 1
            if pending[s] is not None:
                pending[s].wait()
            stage[s, pl.ds(0, nrows)] = (
                jnp.dot(src_ref[...], w8_ref[...],
                        preferred_element_type=jnp.float32)
                * scale
            )
            cp = pltpu.make_async_copy(
                stage.at[s, pl.ds(0, nrows)],
                out_ref.at[pl.ds(row_start, nrows)],
                copy_sems.at[s],
            )
            cp.start()
            pending[s] = cp

        h0 = []
        for j in range(2):
            rows = pl.ds(j * q_per, q_per)
            xcps[j].wait()
            comm_r[0, rows] = xbuf[j].astype(jnp.float8_e4m3fn)
            rr = pltpu.make_async_remote_copy(
                src_ref=comm_r.at[0, rows], dst_ref=comm_r.at[1, rows],
                send_sem=send_r.at[j], recv_sem=recv_r.at[j],
                device_id=(right,), device_id_type=pl.DeviceIdType.MESH,
            )
            rr.start()
            xcps[2 + j].wait()
            comm_l[0, rows] = xbuf[2 + j].astype(jnp.float8_e4m3fn)
            ll = pltpu.make_async_remote_copy(
                src_ref=comm_l.at[0, rows], dst_ref=comm_l.at[1, rows],
                send_sem=send_l.at[j], recv_sem=recv_l.at[j],
                device_id=(left,), device_id_type=pl.DeviceIdType.MESH,
            )
            ll.start()
            h0.append((rr, ll))
        w_piece_copy(0).start()
        stream_w()
        task(comm_r.at[0], my_pos * m_per, h_per)
        task(comm_l.at[0], my_pos * m_per + h_per, h_per)
        for rr, ll in h0:
            rr.wait()
            ll.wait()

        rdma_r = pltpu.make_async_remote_copy(
            src_ref=comm_r.at[1], dst_ref=comm_r.at[2],
            send_sem=send_r.at[2], recv_sem=recv_r.at[2],
            device_id=(right,), device_id_type=pl.DeviceIdType.MESH,
        )
        rdma_l = pltpu.make_async_remote_copy(
            src_ref=comm_l.at[1], dst_ref=comm_l.at[2],
            send_sem=send_l.at[2], recv_sem=recv_l.at[2],
            device_id=(left,), device_id_type=pl.DeviceIdType.MESH,
        )
        rdma_r.start()
        rdma_l.start()
        task(comm_r.at[1], ((my_pos - 1) % N_DEV) * m_per, h_per)
        task(comm_l.at[1], ((my_pos + 1) % N_DEV) * m_per + h_per, h_per)
        rdma_r.wait()
        rdma_l.wait()

        sub = h_per // SUBS
        subs = []
        for j in range(SUBS):
            rows = pl.ds(j * sub, sub)
            rr = pltpu.make_async_remote_copy(
                src_ref=comm_r.at[2, rows], dst_ref=comm_r.at[3, rows],
                send_sem=send_r.at[3 + j], recv_sem=recv_r.at[3 + j],
                device_id=(right,), device_id_type=pl.DeviceIdType.MESH,
            )
            ll = pltpu.make_async_remote_copy(
                src_ref=comm_l.at[2, rows], dst_ref=comm_l.at[3, rows],
                send_sem=send_l.at[3 + j], recv_sem=recv_l.at[3 + j],
                device_id=(left,), device_id_type=pl.DeviceIdType.MESH,
            )
            rr.start()
            ll.start()
            subs.append((rr, ll))
        task(comm_r.at[2], ((my_pos - 2) % N_DEV) * m_per, h_per)
        task(comm_l.at[2], ((my_pos + 2) % N_DEV) * m_per + h_per, h_per)
        orig_r = ((my_pos - 3) % N_DEV) * m_per
        orig_l = ((my_pos + 3) % N_DEV) * m_per + h_per
        for j, (rr, ll) in enumerate(subs):
            rr.wait()
            task(comm_r.at[3, pl.ds(j * sub, sub)], orig_r + j * sub, sub)
            ll.wait()
            task(comm_l.at[3, pl.ds(j * sub, sub)], orig_l + j * sub, sub)
        for cp in pending:
            cp.wait()

    return pl.pallas_call(
        body,
        out_shape=jax.ShapeDtypeStruct((N_DEV * m_per, n_per), jnp.float32),
        in_specs=[
            pl.BlockSpec(memory_space=pl.ANY),
            pl.BlockSpec(memory_space=pl.ANY),
            pl.BlockSpec(memory_space=pltpu.SMEM),
            pl.BlockSpec(memory_space=pltpu.SMEM),
        ],
        out_specs=pl.BlockSpec(memory_space=pl.ANY),
        scratch_shapes=[
            pltpu.VMEM((k, n_per), jnp.float8_e4m3fn),
            pltpu.VMEM((2, W_PIECE, n_per), jnp.float32),
            pltpu.VMEM((4, m_per // 4, k), jnp.float32),
            pltpu.VMEM((N_DEV, h_per, k), jnp.float8_e4m3fn),
            pltpu.VMEM((N_DEV, h_per, k), jnp.float8_e4m3fn),
            pltpu.VMEM((2, h_per, n_per), jnp.float32),
            pltpu.SemaphoreType.DMA((2,)),
            pltpu.SemaphoreType.DMA((4,)),
            pltpu.SemaphoreType.DMA((2,)),
            pltpu.SemaphoreType.DMA((3 + SUBS,)),
            pltpu.SemaphoreType.DMA((3 + SUBS,)),
            pltpu.SemaphoreType.DMA((3 + SUBS,)),
            pltpu.SemaphoreType.DMA((3 + SUBS,)),
        ],
        compiler_params=pltpu.CompilerParams(
            collective_id=0,
            vmem_limit_bytes=63 * 1024 * 1024,
        ),
    )(x, w_mat, scale_x, scale_w)
